# baseline (speedup 1.0000x reference)
import functools

import jax
import jax.numpy as jnp
from jax import lax
from jax.experimental import pallas as pl
from jax.experimental.pallas import tpu as pltpu
from jax.experimental.pallas import tpu_sc as plsc

_NUM_CORES = 2
_NUM_SUBCORES = 16
_NW = _NUM_CORES * _NUM_SUBCORES
_LANES = 16
_SPLITS = 4


@functools.lru_cache(maxsize=None)
def _make_kernel(B, T, D, V):
    assert T % _NW == 0 and D % _LANES == 0
    tpw = T // _NW
    assert tpw % _SPLITS == 0
    q = tpw // _SPLITS
    groups = D // _LANES

    mesh = plsc.VectorSubcoreMesh(core_axis_name="c", subcore_axis_name="s")

    @functools.partial(
        pl.kernel,
        mesh=mesh,
        out_type=jax.ShapeDtypeStruct((B * T, D), jnp.float32),
        scratch_types=[
            pltpu.VMEM((B, tpw), jnp.int32),
            pltpu.VMEM((tpw, D), jnp.float32),
            pltpu.VMEM((tpw, D), jnp.float32),
            pltpu.SemaphoreType.DMA,
            pltpu.SemaphoreType.DMA,
            pltpu.SemaphoreType.DMA((_SPLITS,)),
            pltpu.SemaphoreType.DMA((_SPLITS,)),
        ],
    )
    def emb(x_hbm, tok_hbm, pos_hbm, out_hbm, idx_v, rows_v, pos_v,
            sem_i, sem_p, sem_g, sem_w):
        wid = lax.axis_index("s") * _NUM_CORES + lax.axis_index("c")
        t0 = wid * tpw

        idx_d = [pltpu.async_copy(x_hbm.at[b, pl.ds(t0, tpw)],
                                  idx_v.at[b], sem_i) for b in range(B)]
        pos_d = pltpu.async_copy(pos_hbm.at[pl.ds(t0, tpw)], pos_v, sem_p)

        def add_rows(r_lo, r_hi):
            def row_add(r, carry):
                for g in range(groups):
                    sl = pl.ds(g * _LANES, _LANES)
                    plsc.addupdate(rows_v.at[r, sl], pos_v[r, sl])
                return carry
            lax.fori_loop(r_lo, r_hi, row_add, 0)

        writes = [None] * _SPLITS
        gathers = [None] * _SPLITS
        for b in range(B):
            idx_d[b].wait()
            for s in range(_SPLITS):
                if writes[s] is not None:
                    writes[s].wait()
                gathers[s] = pltpu.async_copy(
                    tok_hbm.at[idx_v.at[b, pl.ds(s * q, q)]],
                    rows_v.at[pl.ds(s * q, q)], sem_g.at[s])
            if b == 0:
                pos_d.wait()
            base = b * T + t0
            for s in range(_SPLITS):
                gathers[s].wait()
                add_rows(s * q, (s + 1) * q)
                writes[s] = pltpu.async_copy(
                    rows_v.at[pl.ds(s * q, q)],
                    out_hbm.at[pl.ds(base + s * q, q)], sem_w.at[s])
        for s in range(_SPLITS):
            writes[s].wait()

    return emb


def kernel(x, tok_table, pos_table):
    B, T = x.shape
    V, D = tok_table.shape
    emb = _make_kernel(B, T, D, V)
    out = emb(x.astype(jnp.int32), tok_table, pos_table)
    return out.reshape(B, T, D)
